# Initial kernel scaffold; baseline (speedup 1.0000x reference)
import jax
import jax.numpy as jnp
from jax.experimental import pallas as pl


def kernel(sim):
    raise NotImplementedError("write your pallas kernel here")


# speedup vs baseline: 1.9818x; 1.9818x over previous
import functools

import jax
import jax.numpy as jnp
from jax.experimental import pallas as pl

_K = 33
_MIN32 = -2147483648


def _tc_body(x_ref, o_ref):
    x = x_ref[...]
    b = jax.lax.bitcast_convert_type(x, jnp.int32)
    skey = b ^ ((b >> 31) & jnp.int32(0x7FFFFFFF))
    rows = x.shape[0]

    def step(i, prefix_u):
        bit = jnp.int32(1) << (jnp.int32(31) - i)
        cand_u = prefix_u | bit
        cand_s = cand_u ^ jnp.int32(_MIN32)
        cnt = jnp.sum((skey >= cand_s).astype(jnp.int32), axis=1,
                      keepdims=True)
        return jnp.where(cnt >= _K, cand_u, prefix_u)

    prefix_u = jax.lax.fori_loop(0, 32, step, jnp.zeros((rows, 1), jnp.int32))
    t_s = prefix_u ^ jnp.int32(_MIN32)

    gt = skey > t_s
    eq = skey == t_s
    need = _K - jnp.sum(gt.astype(jnp.int32), axis=1, keepdims=True)
    n_eq = jnp.sum(eq.astype(jnp.int32), axis=1, keepdims=True)
    cols = jax.lax.broadcasted_iota(jnp.int32, x.shape, 1)

    def tie_search():
        eq_i = eq.astype(jnp.int32)

        def tstep(k, pfx):
            bit = jnp.int32(1) << (jnp.int32(14) - k)
            cand = pfx + bit
            f = jnp.sum(jnp.where(cols < cand, eq_i, 0), axis=1,
                        keepdims=True)
            return jnp.where(f < need, cand, pfx)

        return jax.lax.fori_loop(0, 15, tstep,
                                 jnp.zeros((rows, 1), jnp.int32))

    c_star = jax.lax.cond(
        jnp.any(n_eq > need), tie_search,
        lambda: jnp.full((rows, 1), x.shape[1], jnp.int32))
    sel = gt | (eq & (cols <= c_star))
    o_ref[...] = jnp.where(sel, x, -jnp.inf)


@jax.jit
def kernel(sim):
    n_rows, n_cols = sim.shape
    block_rows = 8
    return pl.pallas_call(
        _tc_body,
        grid=(n_rows // block_rows,),
        in_specs=[pl.BlockSpec((block_rows, n_cols), lambda i: (i, 0))],
        out_specs=pl.BlockSpec((block_rows, n_cols), lambda i: (i, 0)),
        out_shape=jax.ShapeDtypeStruct(sim.shape, sim.dtype),
    )(sim)


# speedup vs baseline: 2.0302x; 1.0244x over previous
import functools

import jax
import jax.numpy as jnp
from jax import lax
from jax.experimental import pallas as pl
from jax.experimental.pallas import tpu as pltpu
from jax.experimental.pallas import tpu_sc as plsc

_K = 33
_MIN32 = -2147483648
_ROWS = 128
_COLS = 32768
_CAP = 4096
_L = 16


def _skey(v):
    b = lax.bitcast_convert_type(v, jnp.int32)
    return b ^ ((b >> 31) & jnp.int32(0x7FFFFFFF))


def _sc_body(sim_hbm, out_hbm, row_v, out_v, candv, candi):
    nc = 2
    wid = lax.axis_index("s") * nc + lax.axis_index("c")
    iota = lax.iota(jnp.int32, _L)
    ninf = jnp.full((_L,), -jnp.inf, jnp.float32)
    n_chunks = _COLS // _L

    def do_row(r):
        pltpu.sync_copy(sim_hbm.at[r], row_v)

        def pass_a(i, carry):
            m1, m2, m3 = carry
            x = row_v[pl.ds(i * _L, _L)]
            hi1 = jnp.maximum(m1, x)
            lo1 = jnp.minimum(m1, x)
            hi2 = jnp.maximum(m2, lo1)
            lo2 = jnp.minimum(m2, lo1)
            return hi1, hi2, jnp.maximum(m3, lo2)

        m1, m2, m3 = lax.fori_loop(0, n_chunks, pass_a, (ninf, ninf, ninf))
        t0 = jnp.broadcast_to(jnp.min(m3), (_L,))

        def pass_b(i, cntv):
            x = row_v[pl.ds(i * _L, _L)]
            msk = x >= t0
            out_v[pl.ds(i * _L, _L)] = jnp.where(msk, x, ninf)
            pos = cntv + plsc.cumsum(msk.astype(jnp.int32)) - 1
            pos = jnp.minimum(pos, _CAP - 1)
            plsc.store_scatter(candv, [pos], x, mask=msk)
            plsc.store_scatter(candi, [pos], iota + i * _L, mask=msk)
            return cntv + plsc.all_reduce_population_count(msk)

        cntv = lax.fori_loop(0, n_chunks, pass_b,
                             jnp.zeros((_L,), jnp.int32))
        cntv = jnp.minimum(cntv, _CAP)
        n_cand = jnp.max(cntv)
        nch = (n_cand + _L - 1) // _L

        def search_step(k, prefix_u):
            bit = jnp.int32(1) << (jnp.int32(31) - k)
            cand_u = prefix_u | bit
            cand_s = jnp.broadcast_to(cand_u ^ jnp.int32(_MIN32), (_L,))

            def count_chunk(ch, acc):
                b0 = ch * _L
                sk = _skey(candv[pl.ds(b0, _L)])
                valid = (iota + b0) < cntv
                return acc + ((sk >= cand_s) & valid).astype(jnp.int32)

            acc = lax.fori_loop(0, nch, count_chunk,
                                jnp.zeros((_L,), jnp.int32))
            return jnp.where(jnp.sum(acc) >= _K, cand_u, prefix_u)

        prefix_u = lax.fori_loop(0, 32, search_step, jnp.int32(0))
        t_v = jnp.broadcast_to(prefix_u ^ jnp.int32(_MIN32), (_L,))

        def gt_chunk(ch, acc):
            b0 = ch * _L
            sk = _skey(candv[pl.ds(b0, _L)])
            valid = (iota + b0) < cntv
            return acc + ((sk > t_v) & valid).astype(jnp.int32)

        n_gt = jnp.sum(lax.fori_loop(0, nch, gt_chunk,
                                     jnp.zeros((_L,), jnp.int32)))
        need = jnp.broadcast_to(_K - n_gt, (_L,))

        def fixup_chunk(ch, rank_base):
            b0 = ch * _L
            sk = _skey(candv[pl.ds(b0, _L)])
            iv = candi[pl.ds(b0, _L)]
            valid = (iota + b0) < cntv
            gtm = (sk > t_v) & valid
            eqm = (sk == t_v) & valid
            rank = jnp.broadcast_to(rank_base, (_L,)) + \
                plsc.cumsum(eqm.astype(jnp.int32))
            keep = gtm | (eqm & (rank <= need))
            rej = valid & jnp.logical_not(keep)
            plsc.store_scatter(out_v, [iv], ninf, mask=rej)
            return rank_base + jnp.sum(eqm.astype(jnp.int32))

        lax.fori_loop(0, nch, fixup_chunk, jnp.int32(0))
        pltpu.sync_copy(out_v, out_hbm.at[r])

    rows_per = _ROWS // 32
    for j in range(rows_per):
        do_row(wid * rows_per + j)


@jax.jit
def kernel(sim):
    mesh = plsc.VectorSubcoreMesh(core_axis_name="c", subcore_axis_name="s")
    f = functools.partial(
        pl.kernel,
        mesh=mesh,
        out_type=jax.ShapeDtypeStruct((_ROWS, _COLS), jnp.float32),
        scratch_types=[
            pltpu.VMEM((_COLS,), jnp.float32),
            pltpu.VMEM((_COLS,), jnp.float32),
            pltpu.VMEM((_CAP,), jnp.float32),
            pltpu.VMEM((_CAP,), jnp.int32),
        ],
        compiler_params=pltpu.CompilerParams(needs_layout_passes=False),
    )(_sc_body)
    return f(sim)


# speedup vs baseline: 2.2522x; 1.1093x over previous
import functools

import jax
import jax.numpy as jnp
from jax import lax
from jax.experimental import pallas as pl
from jax.experimental.pallas import tpu as pltpu
from jax.experimental.pallas import tpu_sc as plsc

_K = 33
_MIN32 = -2147483648
_ROWS = 128
_COLS = 32768
_CAP = 4096
_L = 16
_U = 4


def _skey(v):
    b = lax.bitcast_convert_type(v, jnp.int32)
    return b ^ ((b >> 31) & jnp.int32(0x7FFFFFFF))


def _sc_body(sim_hbm, out_hbm, row_v, out_v, candv, candi):
    nc = 2
    wid = lax.axis_index("s") * nc + lax.axis_index("c")
    iota = lax.iota(jnp.int32, _L)
    ninf = jnp.full((_L,), -jnp.inf, jnp.float32)
    n_chunks = _COLS // _L

    def do_row(r):
        pltpu.sync_copy(sim_hbm.at[r], row_v)

        def pass_a(i, carry):
            out = []
            for u in range(_U):
                m1, m2 = carry[2 * u], carry[2 * u + 1]
                x = row_v[pl.ds((i * _U + u) * _L, _L)]
                hi = jnp.maximum(m1, x)
                lo = jnp.minimum(m1, x)
                out += [hi, jnp.maximum(m2, lo)]
            return tuple(out)

        ms = lax.fori_loop(0, n_chunks // _U, pass_a, (ninf,) * (2 * _U))
        m2min = ms[1]
        for u in range(1, _U):
            m2min = jnp.minimum(m2min, ms[2 * u + 1])
        t0 = jnp.broadcast_to(jnp.min(m2min), (_L,))

        def pass_b(i, cntv):
            for u in range(_U):
                c0 = (i * _U + u) * _L
                x = row_v[pl.ds(c0, _L)]
                msk = x >= t0
                out_v[pl.ds(c0, _L)] = jnp.where(msk, x, ninf)
                pos = cntv + plsc.cumsum(msk.astype(jnp.int32)) - 1
                pos = jnp.minimum(pos, _CAP - 1)
                plsc.store_scatter(candv, [pos], x, mask=msk)
                plsc.store_scatter(candi, [pos], iota + c0, mask=msk)
                cntv = cntv + plsc.all_reduce_population_count(msk)
            return cntv

        cntv = lax.fori_loop(0, n_chunks // _U, pass_b,
                             jnp.zeros((_L,), jnp.int32))
        cntv = jnp.minimum(cntv, _CAP)
        n_cand = jnp.max(cntv)
        nch = (n_cand + _L - 1) // _L

        def search_step(k, prefix_u):
            bit = jnp.int32(1) << (jnp.int32(31) - k)
            cand_u = prefix_u | bit
            cand_s = jnp.broadcast_to(cand_u ^ jnp.int32(_MIN32), (_L,))

            def count_chunk(ch, acc):
                b0 = ch * _L
                sk = _skey(candv[pl.ds(b0, _L)])
                valid = (iota + b0) < cntv
                return acc + ((sk >= cand_s) & valid).astype(jnp.int32)

            acc = lax.fori_loop(0, nch, count_chunk,
                                jnp.zeros((_L,), jnp.int32))
            return jnp.where(jnp.sum(acc) >= _K, cand_u, prefix_u)

        prefix_u = lax.fori_loop(0, 32, search_step, jnp.int32(0))
        t_v = jnp.broadcast_to(prefix_u ^ jnp.int32(_MIN32), (_L,))

        def gt_chunk(ch, acc):
            b0 = ch * _L
            sk = _skey(candv[pl.ds(b0, _L)])
            valid = (iota + b0) < cntv
            return acc + ((sk > t_v) & valid).astype(jnp.int32)

        n_gt = jnp.sum(lax.fori_loop(0, nch, gt_chunk,
                                     jnp.zeros((_L,), jnp.int32)))
        need = jnp.broadcast_to(_K - n_gt, (_L,))

        def fixup_chunk(ch, rank_base):
            b0 = ch * _L
            sk = _skey(candv[pl.ds(b0, _L)])
            iv = candi[pl.ds(b0, _L)]
            valid = (iota + b0) < cntv
            gtm = (sk > t_v) & valid
            eqm = (sk == t_v) & valid
            rank = jnp.broadcast_to(rank_base, (_L,)) + \
                plsc.cumsum(eqm.astype(jnp.int32))
            keep = gtm | (eqm & (rank <= need))
            rej = valid & jnp.logical_not(keep)
            plsc.store_scatter(out_v, [iv], ninf, mask=rej)
            return rank_base + jnp.sum(eqm.astype(jnp.int32))

        lax.fori_loop(0, nch, fixup_chunk, jnp.int32(0))
        pltpu.sync_copy(out_v, out_hbm.at[r])

    rows_per = _ROWS // 32
    for j in range(rows_per):
        do_row(wid * rows_per + j)


@jax.jit
def kernel(sim):
    mesh = plsc.VectorSubcoreMesh(core_axis_name="c", subcore_axis_name="s")
    f = functools.partial(
        pl.kernel,
        mesh=mesh,
        out_type=jax.ShapeDtypeStruct((_ROWS, _COLS), jnp.float32),
        scratch_types=[
            pltpu.VMEM((_COLS,), jnp.float32),
            pltpu.VMEM((_COLS,), jnp.float32),
            pltpu.VMEM((_CAP,), jnp.float32),
            pltpu.VMEM((_CAP,), jnp.int32),
        ],
        compiler_params=pltpu.CompilerParams(needs_layout_passes=False),
    )(_sc_body)
    return f(sim)


# speedup vs baseline: 2.3518x; 1.0442x over previous
import functools

import jax
import jax.numpy as jnp
from jax import lax
from jax.experimental import pallas as pl
from jax.experimental.pallas import tpu as pltpu
from jax.experimental.pallas import tpu_sc as plsc

_K = 33
_MIN32 = -2147483648
_ROWS = 128
_COLS = 32768
_CAP = 4096
_L = 16
_U = 8
_NW = 32


def _skey(v):
    b = lax.bitcast_convert_type(v, jnp.int32)
    return b ^ ((b >> 31) & jnp.int32(0x7FFFFFFF))


def _sc_body(sim_hbm, out_hbm, buf0, buf1, candi, si0, si1, so0, so1):
    nc = 2
    wid = lax.axis_index("s") * nc + lax.axis_index("c")
    iota = lax.iota(jnp.int32, _L)
    ninf = jnp.full((_L,), -jnp.inf, jnp.float32)
    n_chunks = _COLS // _L
    rows_per = _ROWS // _NW
    bufs = [buf0, buf1]
    in_sems = [si0, si1]
    out_sems = [so0, so1]

    def in_copy(j):
        return pltpu.make_async_copy(
            sim_hbm.at[wid * rows_per + j], bufs[j % 2], in_sems[j % 2])

    def out_copy(j):
        return pltpu.make_async_copy(
            bufs[j % 2], out_hbm.at[wid * rows_per + j], out_sems[j % 2])

    def compute(row_v):
        def pass_a(i, carry):
            ms = list(carry)
            for u in range(_U):
                s = u % 4
                x = row_v[pl.ds((i * _U + u) * _L, _L)]
                hi = jnp.maximum(ms[2 * s], x)
                lo = jnp.minimum(ms[2 * s], x)
                ms[2 * s] = hi
                ms[2 * s + 1] = jnp.maximum(ms[2 * s + 1], lo)
            return tuple(ms)

        ms = lax.fori_loop(0, n_chunks // _U, pass_a, (ninf,) * 8)
        m2min = jnp.minimum(jnp.minimum(ms[1], ms[3]),
                            jnp.minimum(ms[5], ms[7]))
        t0 = jnp.broadcast_to(jnp.min(m2min), (_L,))

        def pass_b(i, cntv):
            for u in range(_U):
                c0 = (i * _U + u) * _L
                x = row_v[pl.ds(c0, _L)]
                msk = x >= t0
                row_v[pl.ds(c0, _L)] = jnp.where(msk, x, ninf)
                pos = cntv + plsc.cumsum(msk.astype(jnp.int32)) - 1
                pos = jnp.minimum(pos, _CAP - 1)
                plsc.store_scatter(candi, [pos], iota + c0, mask=msk)
                cntv = cntv + plsc.all_reduce_population_count(msk)
            return cntv

        cntv = lax.fori_loop(0, n_chunks // _U, pass_b,
                             jnp.zeros((_L,), jnp.int32))
        cntv = jnp.minimum(cntv, _CAP)
        n_cand = jnp.max(cntv)
        nch = (n_cand + _L - 1) // _L

        def search_step(k, prefix_u):
            bit = jnp.int32(1) << (jnp.int32(31) - k)
            cand_u = prefix_u | bit
            cand_s = jnp.broadcast_to(cand_u ^ jnp.int32(_MIN32), (_L,))

            def count_chunk(ch, acc):
                b0 = ch * _L
                iv = candi[pl.ds(b0, _L)] & jnp.int32(_COLS - 1)
                sk = _skey(plsc.load_gather(row_v, [iv]))
                valid = (iota + b0) < cntv
                return acc + ((sk >= cand_s) & valid).astype(jnp.int32)

            acc = lax.fori_loop(0, nch, count_chunk,
                                jnp.zeros((_L,), jnp.int32))
            return jnp.where(jnp.sum(acc) >= _K, cand_u, prefix_u)

        prefix_u = lax.fori_loop(0, 32, search_step, jnp.int32(0))
        t_v = jnp.broadcast_to(prefix_u ^ jnp.int32(_MIN32), (_L,))

        def gt_chunk(ch, acc):
            b0 = ch * _L
            iv = candi[pl.ds(b0, _L)] & jnp.int32(_COLS - 1)
            sk = _skey(plsc.load_gather(row_v, [iv]))
            valid = (iota + b0) < cntv
            return acc + ((sk > t_v) & valid).astype(jnp.int32)

        n_gt = jnp.sum(lax.fori_loop(0, nch, gt_chunk,
                                     jnp.zeros((_L,), jnp.int32)))
        need = jnp.broadcast_to(_K - n_gt, (_L,))

        def fixup_chunk(ch, rank_base):
            b0 = ch * _L
            iv = candi[pl.ds(b0, _L)] & jnp.int32(_COLS - 1)
            sk = _skey(plsc.load_gather(row_v, [iv]))
            valid = (iota + b0) < cntv
            gtm = (sk > t_v) & valid
            eqm = (sk == t_v) & valid
            rank = jnp.broadcast_to(rank_base, (_L,)) + \
                plsc.cumsum(eqm.astype(jnp.int32))
            keep = gtm | (eqm & (rank <= need))
            rej = valid & jnp.logical_not(keep)
            plsc.store_scatter(row_v, [iv], ninf, mask=rej)
            return rank_base + jnp.sum(eqm.astype(jnp.int32))

        lax.fori_loop(0, nch, fixup_chunk, jnp.int32(0))

    in_copy(0).start()
    for j in range(rows_per):
        if j + 1 < rows_per:
            if j >= 1:
                out_copy(j - 1).wait()
            in_copy(j + 1).start()
        in_copy(j).wait()
        compute(bufs[j % 2])
        out_copy(j).start()
    out_copy(rows_per - 2).wait()
    out_copy(rows_per - 1).wait()


@jax.jit
def kernel(sim):
    mesh = plsc.VectorSubcoreMesh(core_axis_name="c", subcore_axis_name="s")
    f = functools.partial(
        pl.kernel,
        mesh=mesh,
        out_type=jax.ShapeDtypeStruct((_ROWS, _COLS), jnp.float32),
        scratch_types=[
            pltpu.VMEM((_COLS,), jnp.float32),
            pltpu.VMEM((_COLS,), jnp.float32),
            pltpu.VMEM((_CAP,), jnp.int32),
            pltpu.SemaphoreType.DMA,
            pltpu.SemaphoreType.DMA,
            pltpu.SemaphoreType.DMA,
            pltpu.SemaphoreType.DMA,
        ],
        compiler_params=pltpu.CompilerParams(needs_layout_passes=False),
    )(_sc_body)
    return f(sim)
